# Initial kernel scaffold; baseline (speedup 1.0000x reference)
import jax
import jax.numpy as jnp
from jax.experimental import pallas as pl


def kernel(edge_index, node_feat_0, node_feat_1, edge_feat_0, edge_feat_1, graph_ids, node_emb_0, node_emb_1, edge_emb_0, edge_emb_1, W1, b1, W2, b2, gamma, beta, Wp, bp):
    raise NotImplementedError("write your pallas kernel here")


# speedup vs baseline: 3.9885x; 3.9885x over previous
import jax
import jax.numpy as jnp
from jax import lax
from jax.experimental import pallas as pl
from jax.experimental.pallas import tpu as pltpu
from jax.experimental.pallas import tpu_sc as plsc

_N, _E, _D, _H, _L, _B = 10000, 320000, 128, 256, 5, 64
_NP = 10240
_BLK = 512
_NT = _NP // _BLK
_NC, _NS = 2, 16
_NW = _NC * _NS
_K = 80
_NCH = _E // (_NW * _K)
_KK = 128
_NCHA = 80
_EP = _NW * _NCHA * _KK
_G = 8
_NG = _NCHA // _G
_RPW = _NP // _NS
_CSL = _NP * 16 // _NS
_F32 = jnp.float32
_HI = lax.Precision.HIGHEST


def _agg_body(h_hbm, src_hbm, dst_hbm, out_hbm,
              src_v, dst_v, rows0_v, rows1_v, zrow_v, sem0, sem1, agg_sh):
    c = lax.axis_index("c")
    s = lax.axis_index("s")
    w = s * _NC + c
    srcw = src_hbm.at[w]
    dstw = dst_hbm.at[w]

    zv = jnp.zeros((16,), _F32)

    @pl.loop(0, 16)
    def _zero_rows(i):
        for t in range(_D // 16):
            zrow_v[i, pl.ds(16 * t, 16)] = zv

    @pl.loop(0, _RPW // 16)
    def _zero_agg(k):
        pltpu.sync_copy(zrow_v, agg_sh.at[pl.ds(s * _RPW + k * 16, 16)])

    plsc.subcore_barrier()

    rows = (rows0_v, rows1_v)
    sems = (sem0, sem1)

    @pl.loop(0, _NG)
    def _grp(g):
        pltpu.sync_copy(srcw.at[pl.ds(g * _G, _G)], src_v)
        pltpu.sync_copy(dstw.at[pl.ds(g * _G, _G)], dst_v)
        pltpu.async_copy(h_hbm.at[src_v.at[0]], rows[0], sems[0])
        for jj in range(_G):
            cur = jj % 2
            if jj < _G - 1:
                pltpu.async_copy(h_hbm.at[src_v.at[jj + 1]],
                                 rows[1 - cur], sems[1 - cur])
            pltpu.make_async_copy(h_hbm.at[src_v.at[jj]],
                                  rows[cur], sems[cur]).wait()
            pltpu.sync_copy(rows[cur], agg_sh.at[dst_v.at[jj]], add=True)

    plsc.subcore_barrier()
    pltpu.sync_copy(agg_sh.at[pl.ds(s * _RPW, _RPW)],
                    out_hbm.at[c].at[pl.ds(s * _RPW, _RPW)])


import functools


@functools.lru_cache(maxsize=None)
def _get_agg_call():
    mesh = plsc.VectorSubcoreMesh(core_axis_name="c", subcore_axis_name="s",
                                  num_cores=_NC, num_subcores=_NS)
    return pl.kernel(
        _agg_body,
        out_type=jax.ShapeDtypeStruct((_NC, _NP, _D), _F32),
        mesh=mesh,
        scratch_types=[
            pltpu.VMEM((_G, _KK), jnp.int32),
            pltpu.VMEM((_G, _KK), jnp.int32),
            pltpu.VMEM((_KK, _D), _F32),
            pltpu.VMEM((_KK, _D), _F32),
            pltpu.VMEM((16, _D), _F32),
            pltpu.SemaphoreType.DMA,
            pltpu.SemaphoreType.DMA,
            pltpu.VMEM_SHARED((_NP, _D), _F32),
        ],
    )


def _hist_body(dst_hbm, e0_hbm, e1_hbm, out_hbm,
               dst_v, e0_v, e1_v, idx0_v, idx1_v, ones_v, zbuf_v, csh):
    c = lax.axis_index("c")
    s = lax.axis_index("s")
    w = s * _NC + c
    pltpu.sync_copy(dst_hbm.at[w], dst_v)
    pltpu.sync_copy(e0_hbm.at[w], e0_v)
    pltpu.sync_copy(e1_hbm.at[w], e1_v)

    @pl.loop(0, 1280 // 16)
    def _z(i):
        zbuf_v[pl.ds(16 * i, 16)] = jnp.zeros((16,), _F32)

    for t in range(_K // 16):
        ones_v[pl.ds(16 * t, 16)] = jnp.ones((16,), _F32)

    @pl.loop(0, _CSL // 1280)
    def _zc(k):
        pltpu.sync_copy(zbuf_v, csh.at[pl.ds(s * _CSL + k * 1280, 1280)])

    @pl.loop(0, _NCH)
    def _bi(j):
        for t in range(_K // 16):
            d16 = dst_v[j, pl.ds(16 * t, 16)] * 16
            idx0_v[j, pl.ds(16 * t, 16)] = d16 + e0_v[j, pl.ds(16 * t, 16)]
            idx1_v[j, pl.ds(16 * t, 16)] = d16 + (e1_v[j, pl.ds(16 * t, 16)] + 8)

    plsc.subcore_barrier()

    @pl.loop(0, _NCH)
    def _scat(j):
        pltpu.sync_copy(ones_v, csh.at[idx0_v.at[j]], add=True)
        pltpu.sync_copy(ones_v, csh.at[idx1_v.at[j]], add=True)

    plsc.subcore_barrier()
    pltpu.sync_copy(csh.at[pl.ds(s * _CSL, _CSL)],
                    out_hbm.at[c].at[pl.ds(s * _CSL, _CSL)])


@functools.lru_cache(maxsize=None)
def _get_hist_call():
    mesh = plsc.VectorSubcoreMesh(core_axis_name="c", subcore_axis_name="s",
                                  num_cores=_NC, num_subcores=_NS)
    return pl.kernel(
        _hist_body,
        out_type=jax.ShapeDtypeStruct((_NC, _NP * 16), _F32),
        mesh=mesh,
        scratch_types=[
            pltpu.VMEM((_NCH, _K), jnp.int32),
            pltpu.VMEM((_NCH, _K), jnp.int32),
            pltpu.VMEM((_NCH, _K), jnp.int32),
            pltpu.VMEM((_NCH, _K), jnp.int32),
            pltpu.VMEM((_NCH, _K), jnp.int32),
            pltpu.VMEM((_K,), _F32),
            pltpu.VMEM((1280,), _F32),
            pltpu.VMEM_SHARED((_NP * 16,), _F32),
        ],
    )


def _h0_body(nf0_ref, nf1_ref, emb0_ref, emb1_ref, out_ref):
    nf0 = nf0_ref[0]
    nf1 = nf1_ref[0]
    oh0 = (lax.broadcasted_iota(jnp.int32, (128, _BLK), 0) == nf0).astype(_F32)
    oh1 = (lax.broadcasted_iota(jnp.int32, (8, _BLK), 0) == nf1).astype(_F32)
    m0 = lax.dot_general(oh0, emb0_ref[...], (((0,), (0,)), ((), ())),
                         preferred_element_type=_F32, precision=_HI)
    m1 = lax.dot_general(oh1, emb1_ref[...], (((0,), (0,)), ((), ())),
                         preferred_element_type=_F32, precision=_HI)
    out_ref[...] = m0 + m1


_h0_call = pl.pallas_call(
    _h0_body,
    grid=(_NT,),
    in_specs=[
        pl.BlockSpec((1, 1, _BLK), lambda i: (i, 0, 0)),
        pl.BlockSpec((1, 1, _BLK), lambda i: (i, 0, 0)),
        pl.BlockSpec((128, _D), lambda i: (0, 0)),
        pl.BlockSpec((8, _D), lambda i: (0, 0)),
    ],
    out_specs=pl.BlockSpec((_BLK, _D), lambda i: (i, 0)),
    out_shape=jax.ShapeDtypeStruct((_NP, _D), _F32),
)


def _mlp_body(agg_ref, c_ref, e_ref, w1_ref, b1_ref, w2_ref, b2_ref,
              x2_ref, st_ref):
    i = pl.program_id(0)
    a = agg_ref[0] + agg_ref[1]
    ce = jnp.dot(c_ref[0] + c_ref[1], e_ref[...],
                 preferred_element_type=_F32, precision=_HI)
    x1 = jnp.dot(a + ce, w1_ref[...],
                 preferred_element_type=_F32) + b1_ref[...]
    x1 = jnp.maximum(x1, 0.0)
    x2 = jnp.dot(x1, w2_ref[...],
                 preferred_element_type=_F32) + b2_ref[...]
    x2_ref[...] = x2
    gr = i * _BLK + lax.broadcasted_iota(jnp.int32, (_BLK, 1), 0)
    xm = x2 * (gr < _N).astype(_F32)
    ss = jnp.sum(xm, axis=0, keepdims=True)
    sq = jnp.sum(xm * xm, axis=0, keepdims=True)

    @pl.when(i == 0)
    def _():
        st_ref[0:1, :] = ss
        st_ref[1:2, :] = sq

    @pl.when(i > 0)
    def _():
        st_ref[0:1, :] += ss
        st_ref[1:2, :] += sq


_mlp_call = pl.pallas_call(
    _mlp_body,
    grid=(_NT,),
    in_specs=[
        pl.BlockSpec((_NC, _BLK, _D), lambda i: (0, i, 0)),
        pl.BlockSpec((_NC, _BLK, 16), lambda i: (0, i, 0)),
        pl.BlockSpec((16, _D), lambda i: (0, 0)),
        pl.BlockSpec((_D, _H), lambda i: (0, 0)),
        pl.BlockSpec((1, _H), lambda i: (0, 0)),
        pl.BlockSpec((_H, _D), lambda i: (0, 0)),
        pl.BlockSpec((1, _D), lambda i: (0, 0)),
    ],
    out_specs=[
        pl.BlockSpec((_BLK, _D), lambda i: (i, 0)),
        pl.BlockSpec((2, _D), lambda i: (0, 0)),
    ],
    out_shape=[
        jax.ShapeDtypeStruct((_NP, _D), _F32),
        jax.ShapeDtypeStruct((2, _D), _F32),
    ],
)


def _bn_relu_body(x2_ref, st_ref, g_ref, b_ref, out_ref):
    mu = st_ref[0:1, :] * (1.0 / _N)
    var = st_ref[1:2, :] * (1.0 / _N) - mu * mu
    scale = lax.rsqrt(var + 1e-5) * g_ref[...]
    out_ref[...] = jnp.maximum((x2_ref[...] - mu) * scale + b_ref[...], 0.0)


_bn_relu_call = pl.pallas_call(
    _bn_relu_body,
    grid=(_NT,),
    in_specs=[
        pl.BlockSpec((_BLK, _D), lambda i: (i, 0)),
        pl.BlockSpec((2, _D), lambda i: (0, 0)),
        pl.BlockSpec((1, _D), lambda i: (0, 0)),
        pl.BlockSpec((1, _D), lambda i: (0, 0)),
    ],
    out_specs=pl.BlockSpec((_BLK, _D), lambda i: (i, 0)),
    out_shape=jax.ShapeDtypeStruct((_NP, _D), _F32),
)


def _pool_body(x2_ref, st_ref, g_ref, b_ref, gid_ref, wp_ref,
               out_ref, gsum_v, cnt_v):
    i = pl.program_id(0)
    mu = st_ref[0:1, :] * (1.0 / _N)
    var = st_ref[1:2, :] * (1.0 / _N) - mu * mu
    scale = lax.rsqrt(var + 1e-5) * g_ref[...]
    y = (x2_ref[...] - mu) * scale + b_ref[...]
    gid = gid_ref[0]
    p = (lax.broadcasted_iota(jnp.int32, (_B, _BLK), 0) == gid).astype(_F32)
    gs = jnp.dot(p, y, preferred_element_type=_F32, precision=_HI)
    cn = jnp.sum(p, axis=1, keepdims=True)

    @pl.when(i == 0)
    def _():
        gsum_v[...] = gs
        cnt_v[...] = jnp.broadcast_to(cn, (_B, _D))

    @pl.when(i > 0)
    def _():
        gsum_v[...] += gs
        cnt_v[...] += jnp.broadcast_to(cn, (_B, _D))

    @pl.when(i == _NT - 1)
    def _():
        gf = gsum_v[...] / jnp.maximum(cnt_v[...], 1.0)
        out_ref[...] = jnp.dot(gf, wp_ref[...], preferred_element_type=_F32)


_pool_call = pl.pallas_call(
    _pool_body,
    grid=(_NT,),
    in_specs=[
        pl.BlockSpec((_BLK, _D), lambda i: (i, 0)),
        pl.BlockSpec((2, _D), lambda i: (0, 0)),
        pl.BlockSpec((1, _D), lambda i: (0, 0)),
        pl.BlockSpec((1, _D), lambda i: (0, 0)),
        pl.BlockSpec((1, 1, _BLK), lambda i: (i, 0, 0)),
        pl.BlockSpec((_D, 1), lambda i: (0, 0)),
    ],
    out_specs=pl.BlockSpec((_B, 1), lambda i: (0, 0)),
    out_shape=jax.ShapeDtypeStruct((_B, 1), _F32),
    scratch_shapes=[
        pltpu.VMEM((_B, _D), _F32),
        pltpu.VMEM((_B, _D), _F32),
    ],
)


def kernel(edge_index, node_feat_0, node_feat_1, edge_feat_0, edge_feat_1,
           graph_ids, node_emb_0, node_emb_1, edge_emb_0, edge_emb_1,
           W1, b1, W2, b2, gamma, beta, Wp, bp):
    i32 = jnp.int32
    epad_n = _EP - _E
    pad_src = jnp.zeros((epad_n,), i32)
    pad_dst = _N + (jnp.arange(epad_n, dtype=i32) % (_NP - _N))
    src3a = jnp.concatenate([edge_index[0].astype(i32), pad_src]).reshape(_NW, _NCHA, _KK)
    dst3a = jnp.concatenate([edge_index[1].astype(i32), pad_dst]).reshape(_NW, _NCHA, _KK)
    dst3h = edge_index[1].astype(i32).reshape(_NW, _NCH, _K)
    e02d = edge_feat_0.astype(i32).reshape(_NW, _NCH, _K)
    e12d = edge_feat_1.astype(i32).reshape(_NW, _NCH, _K)
    pad = _NP - _N
    nf0 = jnp.concatenate([node_feat_0.astype(i32),
                           jnp.zeros((pad,), i32)]).reshape(_NT, 1, _BLK)
    nf1 = jnp.concatenate([node_feat_1.astype(i32),
                           jnp.zeros((pad,), i32)]).reshape(_NT, 1, _BLK)
    gid = jnp.concatenate([graph_ids.astype(i32),
                           jnp.full((pad,), _B, i32)]).reshape(_NT, 1, _BLK)
    emb0p = jnp.zeros((128, _D), _F32).at[:120].set(node_emb_0.astype(_F32))
    emb1p = jnp.zeros((8, _D), _F32).at[:3].set(node_emb_1.astype(_F32))
    epad = (jnp.zeros((_L, 16, _D), _F32)
            .at[:, 0:6].set(edge_emb_0.astype(_F32))
            .at[:, 8:11].set(edge_emb_1.astype(_F32)))
    w1t = jnp.swapaxes(W1, 1, 2).astype(_F32)
    w2t = jnp.swapaxes(W2, 1, 2).astype(_F32)
    b1r = b1.reshape(_L, 1, _H).astype(_F32)
    b2r = b2.reshape(_L, 1, _D).astype(_F32)
    gr = gamma.reshape(_L, 1, _D).astype(_F32)
    br = beta.reshape(_L, 1, _D).astype(_F32)
    wpt = Wp.astype(_F32).reshape(1, _D).T

    h = _h0_call(nf0, nf1, emb0p, emb1p)
    chist = _get_hist_call()(dst3h, e02d, e12d).reshape(_NC, _NP, 16)

    out = None
    for l in range(_L):
        agg2 = _get_agg_call()(h, src3a, dst3a)
        x2, st = _mlp_call(agg2, chist, epad[l], w1t[l], b1r[l], w2t[l], b2r[l])
        if l < _L - 1:
            h = _bn_relu_call(x2, st, gr[l], br[l])
        else:
            out = _pool_call(x2, st, gr[l], br[l], gid, wpt)
    return out + bp.astype(_F32).reshape(1, 1)
